# Initial kernel scaffold; baseline (speedup 1.0000x reference)
import functools

import jax
import jax.numpy as jnp
from jax import lax
from jax.experimental import pallas as pl
from jax.experimental.pallas import tpu as pltpu
from jax.experimental.pallas import tpu_sc as plsc

BATCH = 16384
HIST = 200
DIM = 32
TOTAL = BATCH * HIST
NC, NS = 2, 16
NW = NC * NS
PER_W = TOTAL // NW
G = 128
K = 8
CHUNK = G * K
N_CHUNKS = PER_W // CHUNK

_mesh = plsc.VectorSubcoreMesh(
    core_axis_name="c", subcore_axis_name="s", num_cores=NC, num_subcores=NS
)


@functools.partial(
    pl.kernel,
    out_type=jax.ShapeDtypeStruct((TOTAL, DIM), jnp.float32),
    mesh=_mesh,
    scratch_types=[
        pltpu.VMEM((K, G), jnp.int32),
        pltpu.VMEM((CHUNK, DIM), jnp.float32),
        pltpu.SemaphoreType.DMA,
    ],
)
def _gather_kernel(idx_hbm, table_hbm, out_hbm, idx_v, rows_v, sem):
    wid = lax.axis_index("s") * NC + lax.axis_index("c")
    row0 = wid * (PER_W // G)

    @pl.loop(0, N_CHUNKS)
    def _chunk(g):
        koff = row0 + g * K
        pltpu.sync_copy(idx_hbm.at[pl.ds(koff, K)], idx_v)
        copies = [
            pltpu.async_copy(
                table_hbm.at[idx_v.at[j]], rows_v.at[pl.ds(j * G, G)], sem
            )
            for j in range(K)
        ]
        for cpy in copies:
            cpy.wait()
        pltpu.sync_copy(rows_v, out_hbm.at[pl.ds(koff * G, CHUNK)])


def kernel(x, table):
    idx2 = x.reshape(TOTAL // G, G)
    out = _gather_kernel(idx2, table)
    return out.reshape(BATCH, HIST, DIM)


# speedup vs baseline: 6.1742x; 6.1742x over previous
import functools

import jax
import jax.numpy as jnp
from jax import lax
from jax.experimental import pallas as pl
from jax.experimental.pallas import tpu as pltpu
from jax.experimental.pallas import tpu_sc as plsc

BATCH = 16384
HIST = 200
DIM = 32
TOTAL = BATCH * HIST
NC, NS = 2, 16
NW = NC * NS
PER_W = TOTAL // NW
G = 128
K = 8
CHUNK = G * K
N_CHUNKS = PER_W // CHUNK

_mesh = plsc.VectorSubcoreMesh(
    core_axis_name="c", subcore_axis_name="s", num_cores=NC, num_subcores=NS
)


@functools.partial(
    pl.kernel,
    out_type=jax.ShapeDtypeStruct((TOTAL, DIM), jnp.float32),
    mesh=_mesh,
    scratch_types=[
        pltpu.VMEM((K, G), jnp.int32),
        pltpu.VMEM((CHUNK, DIM), jnp.float32),
        pltpu.SemaphoreType.DMA,
    ],
    compiler_params=pltpu.CompilerParams(use_tc_tiling_on_sc=False),
)
def _gather_kernel(idx_hbm, table_hbm, out_hbm, idx_v, rows_v, sem):
    wid = lax.axis_index("s") * NC + lax.axis_index("c")
    row0 = wid * (PER_W // G)

    @pl.loop(0, N_CHUNKS)
    def _chunk(g):
        koff = row0 + g * K
        pltpu.sync_copy(idx_hbm.at[pl.ds(koff, K)], idx_v)
        copies = [
            pltpu.async_copy(
                table_hbm.at[idx_v.at[j]], rows_v.at[pl.ds(j * G, G)], sem
            )
            for j in range(K)
        ]
        for cpy in copies:
            cpy.wait()
        pltpu.sync_copy(rows_v, out_hbm.at[pl.ds(koff * G, CHUNK)])


def kernel(x, table):
    idx2 = x.reshape(TOTAL // G, G)
    out = _gather_kernel(idx2, table)
    return out.reshape(BATCH, HIST, DIM)


# speedup vs baseline: 6.5363x; 1.0586x over previous
import functools

import jax
import jax.numpy as jnp
from jax import lax
from jax.experimental import pallas as pl
from jax.experimental.pallas import tpu as pltpu
from jax.experimental.pallas import tpu_sc as plsc

BATCH = 16384
HIST = 200
DIM = 32
TOTAL = BATCH * HIST
NC, NS = 2, 16
NW = NC * NS
PER_W = TOTAL // NW
G = 128
K = 8
CHUNK = G * K
N_CHUNKS = PER_W // CHUNK
NBUF = 2

_mesh = plsc.VectorSubcoreMesh(
    core_axis_name="c", subcore_axis_name="s", num_cores=NC, num_subcores=NS
)


@functools.partial(
    pl.kernel,
    out_type=jax.ShapeDtypeStruct((TOTAL, DIM), jnp.float32),
    mesh=_mesh,
    scratch_types=[
        pltpu.VMEM((NBUF, K, G), jnp.int32),
        pltpu.VMEM((NBUF, CHUNK, DIM), jnp.float32),
        pltpu.SemaphoreType.DMA,
        pltpu.SemaphoreType.DMA,
        pltpu.SemaphoreType.DMA,
    ],
    compiler_params=pltpu.CompilerParams(use_tc_tiling_on_sc=False),
)
def _gather_kernel(idx_hbm, table_hbm, out_hbm, idx_v, rows_v, idx_sem,
                   gat_sem, out_sem):
    wid = lax.axis_index("s") * NC + lax.axis_index("c")
    row0 = wid * (PER_W // G)

    for b in range(NBUF):
        pltpu.async_copy(
            idx_hbm.at[pl.ds(row0 + b * K, K)], idx_v.at[b], idx_sem
        )

    @pl.loop(0, N_CHUNKS, step=NBUF)
    def _chunks(g):
        for b in range(NBUF):
            c = g + b
            koff = row0 + c * K
            pltpu.make_async_copy(
                idx_hbm.at[pl.ds(koff, K)], idx_v.at[b], idx_sem
            ).wait()
            @pl.when(c >= NBUF)
            def _():
                pltpu.make_async_copy(
                    rows_v.at[b], out_hbm.at[pl.ds(koff * G, CHUNK)], out_sem
                ).wait()

            copies = [
                pltpu.async_copy(
                    table_hbm.at[idx_v.at[b].at[j]],
                    rows_v.at[b].at[pl.ds(j * G, G)],
                    gat_sem,
                )
                for j in range(K)
            ]
            for cpy in copies:
                cpy.wait()
            @pl.when(c + NBUF < N_CHUNKS)
            def _():
                pltpu.async_copy(
                    idx_hbm.at[pl.ds(koff + NBUF * K, K)], idx_v.at[b], idx_sem
                )

            pltpu.async_copy(
                rows_v.at[b], out_hbm.at[pl.ds(koff * G, CHUNK)], out_sem
            )

    for b in range(NBUF):
        pltpu.make_async_copy(
            rows_v.at[b], out_hbm.at[pl.ds(row0 * G, CHUNK)], out_sem
        ).wait()


def kernel(x, table):
    idx2 = x.reshape(TOTAL // G, G)
    out = _gather_kernel(idx2, table)
    return out.reshape(BATCH, HIST, DIM)


# speedup vs baseline: 6.5513x; 1.0023x over previous
import functools

import jax
import jax.numpy as jnp
from jax import lax
from jax.experimental import pallas as pl
from jax.experimental.pallas import tpu as pltpu
from jax.experimental.pallas import tpu_sc as plsc

BATCH = 16384
HIST = 200
DIM = 32
TOTAL = BATCH * HIST
NC, NS = 2, 16
NW = NC * NS
PER_W = TOTAL // NW
G = 128
K = 8
CHUNK = G * K
N_CHUNKS = PER_W // CHUNK
NBUF = 2

_mesh = plsc.VectorSubcoreMesh(
    core_axis_name="c", subcore_axis_name="s", num_cores=NC, num_subcores=NS
)


@functools.partial(
    pl.kernel,
    out_type=jax.ShapeDtypeStruct((TOTAL, DIM), jnp.float32),
    mesh=_mesh,
    scratch_types=[
        pltpu.VMEM((NBUF, K, G), jnp.int32),
        pltpu.VMEM((NBUF, CHUNK, DIM), jnp.float32),
        [pltpu.SemaphoreType.DMA] * NBUF,
        [pltpu.SemaphoreType.DMA] * NBUF,
        [pltpu.SemaphoreType.DMA] * NBUF,
    ],
    compiler_params=pltpu.CompilerParams(use_tc_tiling_on_sc=False),
)
def _gather_kernel(idx_hbm, table_hbm, out_hbm, idx_v, rows_v, idx_sems,
                   gat_sems, out_sems):
    wid = lax.axis_index("s") * NC + lax.axis_index("c")
    row0 = wid * (PER_W // G)

    def wait_idx(buf):
        pltpu.make_async_copy(
            idx_hbm.at[pl.ds(row0, K)], idx_v.at[buf], idx_sems[buf]
        ).wait()

    def wait_out(buf):
        pltpu.make_async_copy(
            rows_v.at[buf], out_hbm.at[pl.ds(row0 * G, CHUNK)], out_sems[buf]
        ).wait()

    def fire_gathers(buf):
        for j in range(K):
            pltpu.async_copy(
                table_hbm.at[idx_v.at[buf].at[j]],
                rows_v.at[buf].at[pl.ds(j * G, G)],
                gat_sems[buf],
            )

    def drain_gathers(buf):
        pltpu.make_async_copy(
            table_hbm.at[pl.ds(0, CHUNK)], rows_v.at[buf], gat_sems[buf]
        ).wait()

    for b in range(NBUF):
        pltpu.async_copy(
            idx_hbm.at[pl.ds(row0 + b * K, K)], idx_v.at[b], idx_sems[b]
        )
    wait_idx(0)
    fire_gathers(0)

    @pl.loop(0, N_CHUNKS, step=NBUF)
    def _chunks(g):
        for b in range(NBUF):
            c = g + b
            koff = row0 + c * K
            @pl.when(c + 1 < N_CHUNKS)
            def _():
                wait_idx(1 - b)
                @pl.when(c >= 1)
                def _():
                    wait_out(1 - b)
                fire_gathers(1 - b)

            drain_gathers(b)
            @pl.when(c + NBUF < N_CHUNKS)
            def _():
                pltpu.async_copy(
                    idx_hbm.at[pl.ds(koff + NBUF * K, K)], idx_v.at[b],
                    idx_sems[b]
                )

            pltpu.async_copy(
                rows_v.at[b], out_hbm.at[pl.ds(koff * G, CHUNK)], out_sems[b]
            )

    for b in range(NBUF):
        wait_out(b)


def kernel(x, table):
    idx2 = x.reshape(TOTAL // G, G)
    out = _gather_kernel(idx2, table)
    return out.reshape(BATCH, HIST, DIM)
